# baseline (speedup 1.0000x reference)
import functools

import jax
import jax.numpy as jnp
from jax import lax
from jax.experimental import pallas as pl
from jax.experimental.pallas import tpu as pltpu
from jax.experimental.pallas import tpu_sc as plsc

_D = 1024
_ROWS = 4 * 4096
_NCH = 16
_CHR = _ROWS // _NCH
_R = 3

_NW = 32
_RPW = _ROWS // _NW
_SCH = 32
_SNCH = _RPW // _SCH
_SR = 3


def _tc_body(x_hbm, o_hbm, *scratch):
    bufs = scratch[:_R]
    sin = scratch[_R:2 * _R]
    sout = scratch[2 * _R:3 * _R]

    def in_copy(k):
        return pltpu.make_async_copy(
            x_hbm.at[pl.ds(k * _CHR, _CHR)], bufs[k % _R], sin[k % _R]
        )

    def out_copy(k):
        return pltpu.make_async_copy(
            bufs[k % _R], o_hbm.at[pl.ds(k * _CHR, _CHR)], sout[k % _R]
        )

    for k in range(_R - 1):
        in_copy(k).start()
    for k in range(_NCH):
        if k + _R - 1 < _NCH:
            if k >= 1:
                out_copy(k - 1).wait()
            in_copy(k + _R - 1).start()
        in_copy(k).wait()
        out_copy(k).start()
    for k in range(_NCH - _R, _NCH):
        out_copy(k).wait()


def _sc_body(x_hbm, o_hbm, *scratch):
    bufs = scratch[:_SR]
    sin = scratch[_SR:2 * _SR]
    sout = scratch[2 * _SR:3 * _SR]
    wid = lax.axis_index("s") * 2 + lax.axis_index("c")
    base = wid * _RPW

    def in_copy(k):
        return pltpu.make_async_copy(
            x_hbm.at[pl.ds(base + k * _SCH, _SCH)], bufs[k % _SR], sin[k % _SR]
        )

    def out_copy(k):
        return pltpu.make_async_copy(
            bufs[k % _SR], o_hbm.at[pl.ds(base + k * _SCH, _SCH)], sout[k % _SR]
        )

    for k in range(_SR - 1):
        in_copy(k).start()
    for k in range(_SNCH):
        if k + _SR - 1 < _SNCH:
            if k >= 1:
                out_copy(k - 1).wait()
            in_copy(k + _SR - 1).start()
        in_copy(k).wait()
        out_copy(k).start()
    for k in range(_SNCH - _SR, _SNCH):
        out_copy(k).wait()


def kernel(x, pe_weight):
    del pe_weight
    b, s, d = x.shape
    x2 = x.reshape(b * s, d)
    tc_out = pl.pallas_call(
        _tc_body,
        out_shape=jax.ShapeDtypeStruct((b * s, d), x.dtype),
        in_specs=[pl.BlockSpec(memory_space=pl.ANY)],
        out_specs=pl.BlockSpec(memory_space=pl.ANY),
        scratch_shapes=(
            [pltpu.VMEM((_CHR, _D), x.dtype) for _ in range(_R)]
            + [pltpu.SemaphoreType.DMA for _ in range(2 * _R)]
        ),
    )(x2)
    mesh = plsc.VectorSubcoreMesh(core_axis_name="c", subcore_axis_name="s")
    sc_copy = functools.partial(
        pl.kernel,
        mesh=mesh,
        out_type=jax.ShapeDtypeStruct((b * s, d), x.dtype),
        scratch_types=(
            [pltpu.VMEM((_SCH, _D), jnp.float32) for _ in range(_SR)]
            + [pltpu.SemaphoreType.DMA for _ in range(2 * _SR)]
        ),
    )(_sc_body)
    sc_out = sc_copy(x2)
    return tc_out.reshape(b, s, d), sc_out.reshape(b, s, d)


# speedup vs baseline: 2.1641x; 2.1641x over previous
import jax
import jax.numpy as jnp
from jax.experimental import pallas as pl
from jax.experimental.pallas import tpu as pltpu

_D = 1024
_ROWS = 4 * 4096
_NCH = 32
_CHR = _ROWS // _NCH
_R = 4


def _copy_body(x_hbm, o_hbm, *scratch):
    bufs = scratch[:_R]
    sin = scratch[_R:2 * _R]
    sout = scratch[2 * _R:3 * _R]

    def in_copy(k):
        return pltpu.make_async_copy(
            x_hbm.at[pl.ds(k * _CHR, _CHR)], bufs[k % _R], sin[k % _R]
        )

    def out_copy(k):
        return pltpu.make_async_copy(
            bufs[k % _R], o_hbm.at[pl.ds(k * _CHR, _CHR)], sout[k % _R]
        )

    for k in range(_R - 1):
        in_copy(k).start()
    for k in range(_NCH):
        if k + _R - 1 < _NCH:
            if k >= 1:
                out_copy(k - 1).wait()
            in_copy(k + _R - 1).start()
        in_copy(k).wait()
        out_copy(k).start()
    for k in range(_NCH - _R, _NCH):
        out_copy(k).wait()


def kernel(x, pe_weight):
    del pe_weight
    b, s, d = x.shape
    x2 = x.reshape(b * s, d)
    out = pl.pallas_call(
        _copy_body,
        out_shape=jax.ShapeDtypeStruct((b * s, d), x.dtype),
        in_specs=[pl.BlockSpec(memory_space=pl.ANY)],
        out_specs=pl.BlockSpec(memory_space=pl.ANY),
        scratch_shapes=(
            [pltpu.VMEM((_CHR, _D), x.dtype) for _ in range(_R)]
            + [pltpu.SemaphoreType.DMA for _ in range(2 * _R)]
        ),
    )(x2)
    return out.reshape(b, s, d)


# speedup vs baseline: 2.3024x; 1.0639x over previous
import jax
import jax.numpy as jnp
from jax.experimental import pallas as pl
from jax.experimental.pallas import tpu as pltpu

_D = 1024
_ROWS = 4 * 4096
_CHUNKS = (256, 256, 512, 1024, 2048, 2048, 2048, 2048, 2048, 2048,
           1024, 512, 256, 256)
_OFFS = tuple(sum(_CHUNKS[:i]) for i in range(len(_CHUNKS)))
_NCH = len(_CHUNKS)
_MAXCH = max(_CHUNKS)
_R = 3


def _copy_body(x_hbm, o_hbm, *scratch):
    bufs = scratch[:_R]
    sin = scratch[_R:2 * _R]
    sout = scratch[2 * _R:3 * _R]

    def in_copy(k):
        return pltpu.make_async_copy(
            x_hbm.at[pl.ds(_OFFS[k], _CHUNKS[k])],
            bufs[k % _R].at[pl.ds(0, _CHUNKS[k])],
            sin[k % _R],
        )

    def out_copy(k):
        return pltpu.make_async_copy(
            bufs[k % _R].at[pl.ds(0, _CHUNKS[k])],
            o_hbm.at[pl.ds(_OFFS[k], _CHUNKS[k])],
            sout[k % _R],
        )

    for k in range(_R - 1):
        in_copy(k).start()
    for k in range(_NCH):
        if k + _R - 1 < _NCH:
            if k >= 1:
                out_copy(k - 1).wait()
            in_copy(k + _R - 1).start()
        in_copy(k).wait()
        out_copy(k).start()
    for k in range(_NCH - _R, _NCH):
        out_copy(k).wait()


def kernel(x, pe_weight):
    del pe_weight
    b, s, d = x.shape
    x2 = x.reshape(b * s, d)
    out = pl.pallas_call(
        _copy_body,
        out_shape=jax.ShapeDtypeStruct((b * s, d), x.dtype),
        in_specs=[pl.BlockSpec(memory_space=pl.ANY)],
        out_specs=pl.BlockSpec(memory_space=pl.ANY),
        scratch_shapes=(
            [pltpu.VMEM((_MAXCH, _D), x.dtype) for _ in range(_R)]
            + [pltpu.SemaphoreType.DMA for _ in range(2 * _R)]
        ),
    )(x2)
    return out.reshape(b, s, d)


# speedup vs baseline: 2.4714x; 1.0734x over previous
import jax
import jax.numpy as jnp
from jax.experimental import pallas as pl
from jax.experimental.pallas import tpu as pltpu

_BLOCK_ROWS = 2048


def _copy_body(x_ref, o_ref):
    o_ref[...] = x_ref[...]


def kernel(x, pe_weight):
    del pe_weight
    b, s, d = x.shape
    x2 = x.reshape(b * s, d)
    rows = b * s
    out = pl.pallas_call(
        _copy_body,
        out_shape=jax.ShapeDtypeStruct((rows, d), x.dtype),
        grid=(rows // _BLOCK_ROWS,),
        in_specs=[pl.BlockSpec((_BLOCK_ROWS, d), lambda i: (i, 0))],
        out_specs=pl.BlockSpec((_BLOCK_ROWS, d), lambda i: (i, 0)),
    )(x2)
    return out.reshape(b, s, d)


# speedup vs baseline: 2.4924x; 1.0085x over previous
import jax
import jax.numpy as jnp
from jax.experimental import pallas as pl
from jax.experimental.pallas import tpu as pltpu

_D = 1024
_ROWS = 4 * 4096
_NCH = 8
_CHR = _ROWS // _NCH
_R = 3


def _copy_body(x_hbm, o_hbm, *scratch):
    bufs = scratch[:_R]
    sin = scratch[_R:2 * _R]
    sout = scratch[2 * _R:3 * _R]

    def in_copy(k):
        return pltpu.make_async_copy(
            x_hbm.at[pl.ds(k * _CHR, _CHR)], bufs[k % _R], sin[k % _R]
        )

    def out_copy(k):
        return pltpu.make_async_copy(
            bufs[k % _R], o_hbm.at[pl.ds(k * _CHR, _CHR)], sout[k % _R]
        )

    for k in range(_R - 1):
        in_copy(k).start()
    for k in range(_NCH):
        if k + _R - 1 < _NCH:
            if k >= 1:
                out_copy(k - 1).wait()
            in_copy(k + _R - 1).start()
        in_copy(k).wait()
        out_copy(k).start()
    for k in range(_NCH - _R, _NCH):
        out_copy(k).wait()


def kernel(x, pe_weight):
    del pe_weight
    b, s, d = x.shape
    x2 = x.reshape(b * s, d)
    out = pl.pallas_call(
        _copy_body,
        out_shape=jax.ShapeDtypeStruct((b * s, d), x.dtype),
        in_specs=[pl.BlockSpec(memory_space=pl.ANY)],
        out_specs=pl.BlockSpec(memory_space=pl.ANY),
        scratch_shapes=(
            [pltpu.VMEM((_CHR, _D), x.dtype) for _ in range(_R)]
            + [pltpu.SemaphoreType.DMA for _ in range(2 * _R)]
        ),
    )(x2)
    return out.reshape(b, s, d)


# speedup vs baseline: 2.5229x; 1.0122x over previous
import jax
import jax.numpy as jnp
from jax.experimental import pallas as pl
from jax.experimental.pallas import tpu as pltpu

_D = 1024
_ROWS = 4 * 4096
_NCH = 8
_CHR = _ROWS // _NCH
_R = 4


def _copy_body(x_hbm, o_hbm, *scratch):
    bufs = scratch[:_R]
    sin = scratch[_R:2 * _R]
    sout = scratch[2 * _R:3 * _R]

    def in_copy(k):
        return pltpu.make_async_copy(
            x_hbm.at[pl.ds(k * _CHR, _CHR)], bufs[k % _R], sin[k % _R]
        )

    def out_copy(k):
        return pltpu.make_async_copy(
            bufs[k % _R], o_hbm.at[pl.ds(k * _CHR, _CHR)], sout[k % _R]
        )

    for k in range(_R - 1):
        in_copy(k).start()
    for k in range(_NCH):
        if k + _R - 1 < _NCH:
            if k >= 1:
                out_copy(k - 1).wait()
            in_copy(k + _R - 1).start()
        in_copy(k).wait()
        out_copy(k).start()
    for k in range(_NCH - _R, _NCH):
        out_copy(k).wait()


def kernel(x, pe_weight):
    del pe_weight
    b, s, d = x.shape
    x2 = x.reshape(b * s, d)
    out = pl.pallas_call(
        _copy_body,
        out_shape=jax.ShapeDtypeStruct((b * s, d), x.dtype),
        in_specs=[pl.BlockSpec(memory_space=pl.ANY)],
        out_specs=pl.BlockSpec(memory_space=pl.ANY),
        scratch_shapes=(
            [pltpu.VMEM((_CHR, _D), x.dtype) for _ in range(_R)]
            + [pltpu.SemaphoreType.DMA for _ in range(2 * _R)]
        ),
    )(x2)
    return out.reshape(b, s, d)
